# Initial kernel scaffold; baseline (speedup 1.0000x reference)
import jax
import jax.numpy as jnp
from jax.experimental import pallas as pl


def kernel(x, W, b):
    raise NotImplementedError("write your pallas kernel here")


# speedup vs baseline: 1.0703x; 1.0703x over previous
import jax
import jax.numpy as jnp
from jax.experimental import pallas as pl
from jax.experimental.pallas import tpu as pltpu

NUM_EXPERTS = 64
TOP_K = 8
BLK_M = 512


def _router_body(x_ref, w_ref, b_ref, wout_ref, iout_ref):
    logits = jax.lax.dot_general(
        x_ref[...].astype(jnp.bfloat16), w_ref[...].astype(jnp.bfloat16),
        (((1,), (1,)), ((), ())),
        preferred_element_type=jnp.float32,
    )
    logits = logits + b_ref[...]
    m = logits.shape[0]
    idxs = jax.lax.broadcasted_iota(jnp.int32, (m, NUM_EXPERTS), 1)
    run = logits
    top_v = []
    top_i = []
    for _ in range(TOP_K):
        mx = jnp.max(run, axis=1, keepdims=True)
        am = jnp.min(jnp.where(run == mx, idxs, NUM_EXPERTS), axis=1,
                     keepdims=True)
        top_v.append(mx)
        top_i.append(am)
        run = jnp.where(idxs == am, -jnp.inf, run)
    vals = jnp.concatenate(top_v, axis=1)
    e = jnp.exp(vals - vals[:, :1])
    w = e / jnp.sum(e, axis=1, keepdims=True)
    wout_ref[...] = w
    iout_ref[...] = jnp.concatenate(top_i, axis=1)


def kernel(x, W, b):
    B, T, d_model = x.shape
    n_tokens = B * T
    xr = x.reshape(n_tokens, d_model)
    grid = (n_tokens // BLK_M,)
    weights, indices = pl.pallas_call(
        _router_body,
        grid=grid,
        in_specs=[
            pl.BlockSpec((BLK_M, d_model), lambda i: (i, 0)),
            pl.BlockSpec((NUM_EXPERTS, d_model), lambda i: (0, 0)),
            pl.BlockSpec((1, NUM_EXPERTS), lambda i: (0, 0)),
        ],
        out_specs=[
            pl.BlockSpec((BLK_M, TOP_K), lambda i: (i, 0)),
            pl.BlockSpec((BLK_M, TOP_K), lambda i: (i, 0)),
        ],
        out_shape=[
            jax.ShapeDtypeStruct((n_tokens, TOP_K), jnp.float32),
            jax.ShapeDtypeStruct((n_tokens, TOP_K), jnp.int32),
        ],
        compiler_params=pltpu.CompilerParams(
            dimension_semantics=("arbitrary",),
        ),
    )(xr, W, b.reshape(1, NUM_EXPERTS))
    aux_loss = jnp.array(0.0, dtype=jnp.float32)
    return (weights.reshape(B, T, TOP_K), indices.reshape(B, T, TOP_K),
            aux_loss)
